# baseline (speedup 1.0000x reference)
import functools

import jax
import jax.numpy as jnp
from jax import lax
from jax.experimental import pallas as pl
from jax.experimental.pallas import tpu as pltpu
from jax.experimental.pallas import tpu_sc as plsc

N = 10000
E = 320000
D = 128

NUM_CORES = 2
NUM_SUBCORES = 16
NW = NUM_CORES * NUM_SUBCORES

SUB = 128
SUPER = 1024
SUBS_PER_SUPER = SUPER // SUB
E_PER_W = 10240
E_PAD = NW * E_PER_W
SUPERS = E_PER_W // SUPER
N_PAD = 10240
ROWS_PER_TILE = N_PAD // NUM_SUBCORES


def _sc_aggregate(x, src, dst2d, w):
    mesh = plsc.VectorSubcoreMesh(core_axis_name="c", subcore_axis_name="s")

    @functools.partial(
        pl.kernel,
        out_type=jax.ShapeDtypeStruct((NUM_CORES, N_PAD, D), jnp.float32),
        mesh=mesh,
        compiler_params=pltpu.CompilerParams(needs_layout_passes=False),
        scratch_types=[
            pltpu.VMEM((SUPER,), jnp.int32),
            pltpu.VMEM((SUPER,), jnp.float32),
            pltpu.VMEM((E_PER_W // 128, 128), jnp.int32),
            pltpu.VMEM((SUB, D), jnp.float32),
            pltpu.VMEM((SUB, D), jnp.float32),
            pltpu.VMEM_SHARED((N_PAD, D), jnp.float32),
            pltpu.SemaphoreType.DMA,
        ],
    )
    def body(x_hbm, src_hbm, dst_hbm, w_hbm, out_hbm, src_v, w_v, dst_v,
             rows_a, rows_b, acc_sh, sem):
        cid = lax.axis_index("c")
        sid = lax.axis_index("s")
        wid = cid * NUM_SUBCORES + sid
        ebase = pl.multiple_of(wid * E_PER_W, E_PER_W)
        bufs = (rows_a, rows_b)

        def _zero_rows(i, _):
            for k in range(D // 16):
                rows_a[i, pl.ds(k * 16, 16)] = jnp.zeros((16,), jnp.float32)
            return 0
        lax.fori_loop(0, SUB, _zero_rows, 0)
        r0 = pl.multiple_of(sid * ROWS_PER_TILE, ROWS_PER_TILE)
        for z in range(ROWS_PER_TILE // SUB):
            pltpu.sync_copy(rows_a, acc_sh.at[pl.ds(r0 + z * SUB, SUB)])

        dstbase = pl.multiple_of(wid * (E_PER_W // 128), E_PER_W // 128)
        pltpu.sync_copy(dst_hbm.at[pl.ds(dstbase, E_PER_W // 128)], dst_v)
        plsc.subcore_barrier()

        PARTS = 4
        PART = SUB // PARTS

        def _gather_start(k, buf):
            for p in range(PARTS):
                pltpu.make_async_copy(
                    x_hbm.at[src_v.at[pl.ds(k * SUB + p * PART, PART)]],
                    buf.at[pl.ds(p * PART, PART)], sem).start()

        def _gather_wait(k, buf):
            for p in range(PARTS):
                pltpu.make_async_copy(
                    x_hbm.at[src_v.at[pl.ds(k * SUB + p * PART, PART)]],
                    buf.at[pl.ds(p * PART, PART)], sem).wait()

        def _scale_buf(buf, k):
            def _scale(i, _):
                wsplat = plsc.load_gather(
                    w_v, [jnp.full((16,), k * SUB + i, jnp.int32)])
                for q in range(D // 16):
                    sl = pl.ds(q * 16, 16)
                    buf[i, sl] = buf[i, sl] * wsplat
                return 0
            lax.fori_loop(0, SUB, _scale, 0)

        def _super(g, _):
            e0 = ebase + g * SUPER
            pltpu.sync_copy(src_hbm.at[pl.ds(e0, SUPER)], src_v)
            pltpu.sync_copy(w_hbm.at[pl.ds(e0, SUPER)], w_v)

            _gather_start(0, rows_a)
            for s in range(SUBS_PER_SUPER):
                buf = bufs[s % 2]
                _gather_wait(s, buf)
                if s + 1 < SUBS_PER_SUPER:
                    _gather_start(s + 1, bufs[(s + 1) % 2])
                _scale_buf(buf, s)
                pltpu.sync_copy(
                    buf, acc_sh.at[dst_v.at[g * SUBS_PER_SUPER + s]], add=True)
            return 0
        lax.fori_loop(0, SUPERS, _super, 0)

        plsc.subcore_barrier()

        @pl.when(cid == 0)
        def _():
            pltpu.sync_copy(acc_sh.at[pl.ds(r0, ROWS_PER_TILE)],
                            out_hbm.at[0, pl.ds(r0, ROWS_PER_TILE)])

        @pl.when(cid == 1)
        def _():
            pltpu.sync_copy(acc_sh.at[pl.ds(r0, ROWS_PER_TILE)],
                            out_hbm.at[1, pl.ds(r0, ROWS_PER_TILE)])

    return body(x, src, dst2d, w)


def _tc_linear(p, W, b2d):
    BLK = 1000

    def body(p_ref, w_ref, b_ref, o_ref):
        acc = p_ref[0] + p_ref[1]
        o_ref[...] = lax.dot_general(
            acc, w_ref[...], (((1,), (1,)), ((), ())),
            preferred_element_type=jnp.float32) + b_ref[...]

    return pl.pallas_call(
        body,
        grid=(N // BLK,),
        in_specs=[
            pl.BlockSpec((NUM_CORES, BLK, D), lambda i: (0, i, 0)),
            pl.BlockSpec((D, D), lambda i: (0, 0)),
            pl.BlockSpec((1, D), lambda i: (0, 0)),
        ],
        out_specs=pl.BlockSpec((BLK, D), lambda i: (i, 0)),
        out_shape=jax.ShapeDtypeStruct((N, D), jnp.float32),
    )(p, W, b2d)


@jax.jit
def kernel(x, edge_index, edge_weight, W, b):
    dst = edge_index[0].astype(jnp.int32)
    src = edge_index[1].astype(jnp.int32)
    pad = E_PAD - E
    src = jnp.concatenate([src, jnp.zeros((pad,), jnp.int32)])
    dst = jnp.concatenate([dst, jnp.zeros((pad,), jnp.int32)])
    w = jnp.concatenate([edge_weight, jnp.zeros((pad,), jnp.float32)])
    dst2d = dst.reshape(E_PAD // 128, 128)

    p = _sc_aggregate(x, src, dst2d, w)
    return _tc_linear(p, W, b.reshape(1, D))


# speedup vs baseline: 1.4408x; 1.4408x over previous
import functools

import jax
import jax.numpy as jnp
from jax import lax
from jax.experimental import pallas as pl
from jax.experimental.pallas import tpu as pltpu
from jax.experimental.pallas import tpu_sc as plsc

N = 10000
E = 320000
D = 128
DW = D // 2

NUM_CORES = 2
NUM_SUBCORES = 16
NW = NUM_CORES * NUM_SUBCORES

SUB = 128
SUPER = 1024
SUBS_PER_SUPER = SUPER // SUB
E_PER_W = 10240
E_PAD = NW * E_PER_W
SUPERS = E_PER_W // SUPER
N_PAD = 10240
ROWS_PER_TILE = N_PAD // NUM_SUBCORES


def _sc_aggregate(xpk, src, dst2d, w):
    mesh = plsc.VectorSubcoreMesh(core_axis_name="c", subcore_axis_name="s")

    @functools.partial(
        pl.kernel,
        out_type=jax.ShapeDtypeStruct((NUM_CORES, N_PAD, D), jnp.float32),
        mesh=mesh,
        compiler_params=pltpu.CompilerParams(
            needs_layout_passes=False, use_tc_tiling_on_sc=False),
        scratch_types=[
            pltpu.VMEM((SUPER,), jnp.int32),
            pltpu.VMEM((SUPER,), jnp.float32),
            pltpu.VMEM((E_PER_W // 128, 128), jnp.int32),
            pltpu.VMEM((SUB, DW), jnp.int32),
            pltpu.VMEM((SUB, DW), jnp.int32),
            pltpu.VMEM((SUB, D), jnp.float32),
            pltpu.VMEM_SHARED((N_PAD, D), jnp.float32),
            pltpu.SemaphoreType.DMA,
        ],
    )
    def body(x_hbm, src_hbm, dst_hbm, w_hbm, out_hbm, src_v, w_v, dst_v,
             rows_a, rows_b, scaled_v, acc_sh, sem):
        cid = lax.axis_index("c")
        sid = lax.axis_index("s")
        wid = cid * NUM_SUBCORES + sid
        ebase = pl.multiple_of(wid * E_PER_W, E_PER_W)
        bufs = (rows_a, rows_b)

        def _zero_rows(i, _):
            for k in range(D // 16):
                scaled_v[i, pl.ds(k * 16, 16)] = jnp.zeros((16,), jnp.float32)
            return 0
        lax.fori_loop(0, SUB, _zero_rows, 0)
        r0 = pl.multiple_of(sid * ROWS_PER_TILE, ROWS_PER_TILE)
        for z in range(ROWS_PER_TILE // SUB):
            pltpu.sync_copy(scaled_v, acc_sh.at[pl.ds(r0 + z * SUB, SUB)])

        dstbase = pl.multiple_of(wid * (E_PER_W // 128), E_PER_W // 128)
        pltpu.sync_copy(dst_hbm.at[pl.ds(dstbase, E_PER_W // 128)], dst_v)
        plsc.subcore_barrier()

        def _gather(k, buf):
            return pltpu.make_async_copy(
                x_hbm.at[src_v.at[pl.ds(k * SUB, SUB)]], buf, sem)

        def _scale_buf(buf, k):
            def _scale(i, _):
                wsplat = plsc.load_gather(
                    w_v, [jnp.full((16,), k * SUB + i, jnp.int32)])
                for q in range(DW // 16):
                    pk = buf[i, pl.ds(q * 16, 16)]
                    pb = plsc.bitcast(pk, jnp.bfloat16)
                    ev, od = plsc.unpack(
                        pb, format=plsc.PackFormat.INTERLEAVED,
                        preferred_element_type=jnp.float32)
                    scaled_v[i, pl.ds(q * 16, 16)] = ev * wsplat
                    scaled_v[i, pl.ds(DW + q * 16, 16)] = od * wsplat
                return 0
            lax.fori_loop(0, SUB, _scale, 0)

        def _super(g, _):
            e0 = ebase + g * SUPER
            pltpu.sync_copy(src_hbm.at[pl.ds(e0, SUPER)], src_v)
            pltpu.sync_copy(w_hbm.at[pl.ds(e0, SUPER)], w_v)

            _gather(0, rows_a).start()
            for s in range(SUBS_PER_SUPER):
                buf = bufs[s % 2]
                _gather(s, buf).wait()
                if s + 1 < SUBS_PER_SUPER:
                    _gather(s + 1, bufs[(s + 1) % 2]).start()
                _scale_buf(buf, s)
                pltpu.sync_copy(
                    scaled_v, acc_sh.at[dst_v.at[g * SUBS_PER_SUPER + s]],
                    add=True)
            return 0
        lax.fori_loop(0, SUPERS, _super, 0)

        plsc.subcore_barrier()

        @pl.when(cid == 0)
        def _():
            pltpu.sync_copy(acc_sh.at[pl.ds(r0, ROWS_PER_TILE)],
                            out_hbm.at[0, pl.ds(r0, ROWS_PER_TILE)])

        @pl.when(cid == 1)
        def _():
            pltpu.sync_copy(acc_sh.at[pl.ds(r0, ROWS_PER_TILE)],
                            out_hbm.at[1, pl.ds(r0, ROWS_PER_TILE)])

    return body(xpk, src, dst2d, w)


def _tc_linear(p, Wp, b2d):
    BLK = 1000

    def body(p_ref, w_ref, b_ref, o_ref):
        acc = p_ref[0] + p_ref[1]
        o_ref[...] = lax.dot_general(
            acc, w_ref[...], (((1,), (1,)), ((), ())),
            preferred_element_type=jnp.float32) + b_ref[...]

    return pl.pallas_call(
        body,
        grid=(N // BLK,),
        in_specs=[
            pl.BlockSpec((NUM_CORES, BLK, D), lambda i: (0, i, 0)),
            pl.BlockSpec((D, D), lambda i: (0, 0)),
            pl.BlockSpec((1, D), lambda i: (0, 0)),
        ],
        out_specs=pl.BlockSpec((BLK, D), lambda i: (i, 0)),
        out_shape=jax.ShapeDtypeStruct((N, D), jnp.float32),
    )(p, Wp, b2d)


@jax.jit
def kernel(x, edge_index, edge_weight, W, b):
    dst = edge_index[0].astype(jnp.int32)
    src = edge_index[1].astype(jnp.int32)
    pad = E_PAD - E
    src = jnp.concatenate([src, jnp.zeros((pad,), jnp.int32)])
    dst = jnp.concatenate([dst, jnp.zeros((pad,), jnp.int32)])
    w = jnp.concatenate([edge_weight, jnp.zeros((pad,), jnp.float32)])
    dst2d = dst.reshape(E_PAD // 128, 128)

    xpk = lax.bitcast_convert_type(
        x.astype(jnp.bfloat16).reshape(N, DW, 2), jnp.int32)
    cols = jnp.concatenate([jnp.arange(0, D, 2), jnp.arange(1, D, 2)])
    Wp = W[:, cols]

    p = _sc_aggregate(xpk, src, dst2d, w)
    return _tc_linear(p, Wp, b.reshape(1, D))
